# baseline (speedup 1.0000x reference)
import functools

import jax
import jax.numpy as jnp
from jax import lax
from jax.experimental import pallas as pl
from jax.experimental.pallas import tpu as pltpu
from jax.experimental.pallas import tpu_sc as plsc

B = 4
C = 96
H = 384
W = 384
HW = H * W
T = 24576
L = 16
CG = C // L
KP = 8
NTAP = 9
NRUN = 10


def _prep_body(seg_ref, w1_ref, w2_ref, fmap_ref, s_ref, sum_ref, max_ref):
    t = pl.program_id(1)
    nt = pl.num_programs(1)
    blk = seg_ref[0].reshape(C, T)
    fmap_ref[...] = jnp.concatenate(
        [blk.T, jnp.zeros((T, 128 - C), jnp.float32)], axis=1)
    psum = jnp.sum(blk, axis=1)[None, :]
    pmax = jnp.max(blk, axis=1)[None, :]

    @pl.when(t == 0)
    def _init():
        sum_ref[...] = psum
        max_ref[...] = pmax

    @pl.when(t != 0)
    def _acc():
        sum_ref[...] += psum
        max_ref[...] = jnp.maximum(max_ref[...], pmax)

    @pl.when(t == nt - 1)
    def _mlp():
        avg = sum_ref[...] * (1.0 / HW)
        mxv = max_ref[...]
        w1 = w1_ref[...]
        w2 = w2_ref[...]

        def fc(v):
            hid = lax.dot_general(v, w1, (((1,), (1,)), ((), ())),
                                  preferred_element_type=jnp.float32)
            hid = jnp.maximum(hid, 0.0)
            return lax.dot_general(hid, w2, (((1,), (1,)), ((), ())),
                                   preferred_element_type=jnp.float32)

        s_ref[0] = jax.nn.sigmoid(fc(avg) + fc(mxv))


def _tc_prep(segment, ca_w1, ca_w2):
    b = segment.shape[0]
    nt = HW // T
    rows = T // W
    return pl.pallas_call(
        _prep_body,
        grid=(b, nt),
        in_specs=[
            pl.BlockSpec((1, C, rows, W), lambda i, t: (i, 0, t, 0)),
            pl.BlockSpec((C, C), lambda i, t: (0, 0)),
            pl.BlockSpec((C, C), lambda i, t: (0, 0)),
        ],
        out_specs=[
            pl.BlockSpec((T, 128), lambda i, t: (i * nt + t, 0)),
            pl.BlockSpec((1, 1, C), lambda i, t: (i, 0, 0)),
        ],
        out_shape=[
            jax.ShapeDtypeStruct((b * HW, 128), jnp.float32),
            jax.ShapeDtypeStruct((b, 1, C), jnp.float32),
        ],
        scratch_shapes=[
            pltpu.VMEM((1, C), jnp.float32),
            pltpu.VMEM((1, C), jnp.float32),
        ],
    )(segment, ca_w1, ca_w2)


def _sc_combine(table, xs, ys, par, lwp, sperm, m):
    info = plsc.get_sparse_core_info()
    nc, ns = info.num_cores, info.num_subcores
    nw = nc * ns
    slots_w = m // nw
    n_chunks = slots_w // KP
    ncw = 32
    nrb = B * NRUN * KP
    mesh = plsc.VectorSubcoreMesh(core_axis_name="c", subcore_axis_name="s")

    @functools.partial(
        pl.kernel,
        mesh=mesh,
        out_type=jax.ShapeDtypeStruct((B * m * C,), jnp.float32),
        scratch_types=[
            pltpu.VMEM((NRUN * B * ncw,), jnp.float32),
            pltpu.VMEM((NRUN * B * ncw,), jnp.float32),
            pltpu.VMEM((2 * nrb,), jnp.int32),
            pltpu.VMEM((2 * nrb, 128), jnp.float32),
            pltpu.VMEM((B * KP * C,), jnp.float32),
            pltpu.VMEM((11 * L,), jnp.float32),
            pltpu.VMEM((B * C,), jnp.float32),
            pltpu.VMEM((B * C,), jnp.float32),
            pltpu.SemaphoreType.DMA,
        ],
        compiler_params=pltpu.CompilerParams(use_tc_tiling_on_sc=False,
                                             needs_layout_passes=False),
    )
    def body(table_h, xs_h, ys_h, par_h, lwp_h, sperm_h, out_h,
             xc_v, yc_v, idx_v, rows_v, out_v, par_v, lwp_v, sp_v, sem):
        wid = lax.axis_index("s") * nc + lax.axis_index("c")
        base0 = wid * slots_w
        pltpu.sync_copy(par_h, par_v)
        pltpu.sync_copy(lwp_h, lwp_v)
        pltpu.sync_copy(sperm_h, sp_v)

        mi_al = []
        for a in range(NRUN):
            lo = ((a * m + base0) // NTAP) // 8 * 8
            mi_al.append(lo)
            for b2 in range(B):
                off = (a * B + b2) * ncw
                pltpu.sync_copy(xs_h.at[pl.ds(b2 * m + lo, ncw)],
                                xc_v.at[pl.ds(off, ncw)])
                pltpu.sync_copy(ys_h.at[pl.ds(b2 * m + lo, ncw)],
                                yc_v.at[pl.ds(off, ncw)])

        cw = [par_v[pl.ds(a * L, L)] for a in range(NTAP)]
        cbv = par_v[pl.ds(9 * L, L)]
        lbv = par_v[pl.ds(10 * L, L)]
        lane = lax.iota(jnp.int32, L)
        mask8 = lane < 8
        half = lane // 8
        sl8 = lane - half * 8
        pre = []
        for j in range(CG):
            cc2 = j * L + lane
            bbj = cc2 // 24
            pre.append(bbj * (KP * C) + (cc2 - 24 * bbj) * 4)

        def fire(ci, par):
            mm0 = base0 + ci * KP
            pbase = par * nrb
            for b2 in range(B):
                for ah in range(NRUN // 2):
                    a0 = 2 * ah
                    aa = a0 + half
                    iv = aa * m + mm0 + sl8
                    mi = iv // NTAP
                    k = iv - mi * NTAP
                    mialv = jnp.where(mask8, mi_al[a0], mi_al[a0 + 1])
                    gi = mi - mialv + aa * (B * ncw) + b2 * ncw
                    xf = plsc.load_gather(xc_v, [gi])
                    yf = plsc.load_gather(yc_v, [gi])
                    xi = (xf * float(H)).astype(jnp.int32)
                    yi = (yf * float(W)).astype(jnp.int32)
                    xo = jnp.clip(xi + (k // 3 - 1), 0, H - 1)
                    yo = jnp.clip(yi + (k - k // 3 * 3 - 1), 0, W - 1)
                    idx_v[pl.ds(pbase + (b2 * NRUN + a0) * KP, L)] = (
                        xo * W + yo + b2 * HW)
            for b2 in range(B):
                boff = b2 * NRUN * KP
                pltpu.async_copy(
                    table_h.at[idx_v.at[pl.ds(pbase + boff, NRUN * KP)]],
                    rows_v.at[pl.ds(pbase + boff, NRUN * KP)], sem)

        fire(0, 0)

        def chunk_body(ci, carry):
            par = lax.rem(ci, 2)
            pbase = par * nrb
            mm0 = base0 + ci * KP

            @pl.when(ci + 1 < n_chunks)
            def _prefetch():
                fire(ci + 1, lax.rem(ci + 1, 2))

            for b2 in range(B):
                boff = b2 * NRUN * KP
                pltpu.make_async_copy(
                    table_h.at[pl.ds(0, NRUN * KP)],
                    rows_v.at[pl.ds(pbase + boff, NRUN * KP)],
                    sem).wait()

            def slot_body(t, inner):
                acc = [None] * B
                def addin(t_, v):
                    acc[t_] = v if acc[t_] is None else acc[t_] + v
                for b2 in range(B):
                    row = pbase + (b2 * NRUN + 4) * KP + t
                    for j in range(CG):
                        cen = rows_v[row, pl.ds(j * L, L)]
                        p = cen * lwp_v[pl.ds(b2 * C + j * L, L)]
                        band = (j * L) // 24
                        if j * L % 24 == 0 and (j * L + L) <= 24 * (band + 1):
                            addin(band, p)
                        elif (j * L + L) <= 24 * (band + 1):
                            addin(band, p)
                        else:
                            plo = jnp.where(mask8, p, 0.0)
                            addin(band, plo)
                            addin(band + 1, p - plo)
                acc = [a_ + lbv * (1.0 / L) for a_ in acc]
                conf = [jnp.sum(a_) for a_ in acc]
                cf = [jnp.full((L,), 1.0 - cs, jnp.float32) for cs in conf]
                coeff = [cf[0], jnp.where(mask8, cf[0], cf[1]), cf[1],
                         cf[2], jnp.where(mask8, cf[2], cf[3]), cf[3]]
                tof = t * C
                for b2 in range(B):
                    for j in range(CG):
                        pool = cbv
                        cen = None
                        for a in range(NTAP):
                            r = rows_v[pbase + (b2 * NRUN + a) * KP + t,
                                       pl.ds(j * L, L)]
                            if a == 4:
                                cen = r
                            pool = pool + cw[a] * r
                        val = (coeff[j] * pool + cen
                               + sp_v[pl.ds(b2 * C + j * L, L)])
                        plsc.store_scatter(out_v, [pre[j] + (tof + b2)], val)
                return inner

            lax.fori_loop(0, KP, slot_body, 0)
            for b2 in range(B):
                pltpu.sync_copy(
                    out_v.at[pl.ds(b2 * KP * C, KP * C)],
                    out_h.at[pl.ds((b2 * m + mm0) * C, KP * C)])
            return carry

        lax.fori_loop(0, n_chunks, chunk_body, 0)

    return body(table, xs, ys, par, lwp, sperm)


def kernel(original_kpts, segment, conv_w, conv_b, lin_w, lin_b, ca_w1, ca_w2):
    b, c, h, w = segment.shape
    m = original_kpts.shape[1]
    table, s3 = _tc_prep(segment, ca_w1, ca_w2)
    s = s3.reshape(b, c)
    pad = jnp.zeros((512,), jnp.float32)
    xs = jnp.concatenate([original_kpts[..., 0].reshape(-1), pad])
    ys = jnp.concatenate([original_kpts[..., 1].reshape(-1), pad])
    par = jnp.concatenate([
        jnp.repeat(conv_w.reshape(9), L),
        jnp.full((L,), conv_b[0], jnp.float32),
        jnp.full((L,), lin_b[0], jnp.float32),
    ])
    cb4 = c // b
    lwp = jnp.tile(lin_w.reshape(cb4, b).T, (1, b)).reshape(-1)
    sperm = jnp.transpose(s.reshape(b, cb4, b), (2, 0, 1)).reshape(-1)
    flat = _sc_combine(table, xs, ys, par, lwp, sperm, m)
    return flat.reshape(b, m, c)


# speedup vs baseline: 1.0813x; 1.0813x over previous
import functools

import jax
import jax.numpy as jnp
from jax import lax
from jax.experimental import pallas as pl
from jax.experimental.pallas import tpu as pltpu
from jax.experimental.pallas import tpu_sc as plsc

B = 4
C = 96
H = 384
W = 384
HW = H * W
T = 24576
L = 16
CG = C // L
KP = 16
NTAP = 9


def _prep_body(seg_ref, w1_ref, w2_ref, fmap_ref, s_ref, sum_ref, max_ref):
    t = pl.program_id(1)
    nt = pl.num_programs(1)
    blk = seg_ref[0].reshape(C, T)
    fmap_ref[...] = jnp.concatenate(
        [blk.T, jnp.zeros((T, 128 - C), jnp.float32)], axis=1)
    psum = jnp.sum(blk, axis=1)[None, :]
    pmax = jnp.max(blk, axis=1)[None, :]

    @pl.when(t == 0)
    def _init():
        sum_ref[...] = psum
        max_ref[...] = pmax

    @pl.when(t != 0)
    def _acc():
        sum_ref[...] += psum
        max_ref[...] = jnp.maximum(max_ref[...], pmax)

    @pl.when(t == nt - 1)
    def _mlp():
        avg = sum_ref[...] * (1.0 / HW)
        mxv = max_ref[...]
        w1 = w1_ref[...]
        w2 = w2_ref[...]

        def fc(v):
            hid = lax.dot_general(v, w1, (((1,), (1,)), ((), ())),
                                  preferred_element_type=jnp.float32)
            hid = jnp.maximum(hid, 0.0)
            return lax.dot_general(hid, w2, (((1,), (1,)), ((), ())),
                                   preferred_element_type=jnp.float32)

        s_ref[0] = jax.nn.sigmoid(fc(avg) + fc(mxv))


def _tc_prep(segment, ca_w1, ca_w2):
    b = segment.shape[0]
    nt = HW // T
    rows = T // W
    return pl.pallas_call(
        _prep_body,
        grid=(b, nt),
        in_specs=[
            pl.BlockSpec((1, C, rows, W), lambda i, t: (i, 0, t, 0)),
            pl.BlockSpec((C, C), lambda i, t: (0, 0)),
            pl.BlockSpec((C, C), lambda i, t: (0, 0)),
        ],
        out_specs=[
            pl.BlockSpec((T, 128), lambda i, t: (i * nt + t, 0)),
            pl.BlockSpec((1, 1, C), lambda i, t: (i, 0, 0)),
        ],
        out_shape=[
            jax.ShapeDtypeStruct((b * HW, 128), jnp.float32),
            jax.ShapeDtypeStruct((b, 1, C), jnp.float32),
        ],
        scratch_shapes=[
            pltpu.VMEM((1, C), jnp.float32),
            pltpu.VMEM((1, C), jnp.float32),
        ],
    )(segment, ca_w1, ca_w2)


def _sc_combine(table, xs, ys, par, lwp, sperm, m):
    info = plsc.get_sparse_core_info()
    nc, ns = info.num_cores, info.num_subcores
    nw = nc * ns
    slots_w = m // nw
    n_chunks = slots_w // KP
    ncw = 32
    mesh = plsc.VectorSubcoreMesh(core_axis_name="c", subcore_axis_name="s")

    @functools.partial(
        pl.kernel,
        mesh=mesh,
        out_type=jax.ShapeDtypeStruct((B * m * C,), jnp.float32),
        scratch_types=[
            pltpu.VMEM((NTAP * B * ncw,), jnp.float32),
            pltpu.VMEM((NTAP * B * ncw,), jnp.float32),
            pltpu.VMEM((NTAP * B * KP,), jnp.int32),
            pltpu.VMEM((NTAP * B * KP, 128), jnp.float32),
            pltpu.VMEM((B * KP * C,), jnp.float32),
            pltpu.VMEM((11 * L,), jnp.float32),
            pltpu.VMEM((B * C,), jnp.float32),
            pltpu.VMEM((B * C,), jnp.float32),
            pltpu.SemaphoreType.DMA,
        ],
        compiler_params=pltpu.CompilerParams(use_tc_tiling_on_sc=False,
                                             needs_layout_passes=False),
    )
    def body(table_h, xs_h, ys_h, par_h, lwp_h, sperm_h, out_h,
             xc_v, yc_v, idx_v, rows_v, out_v, par_v, lwp_v, sp_v, sem):
        wid = lax.axis_index("s") * nc + lax.axis_index("c")
        base0 = wid * slots_w
        pltpu.sync_copy(par_h, par_v)
        pltpu.sync_copy(lwp_h, lwp_v)
        pltpu.sync_copy(sperm_h, sp_v)

        mi_al = []
        for a in range(NTAP):
            lo = ((a * m + base0) // NTAP) // 8 * 8
            mi_al.append(lo)
            for b2 in range(B):
                off = (a * B + b2) * ncw
                pltpu.sync_copy(xs_h.at[pl.ds(b2 * m + lo, ncw)],
                                xc_v.at[pl.ds(off, ncw)])
                pltpu.sync_copy(ys_h.at[pl.ds(b2 * m + lo, ncw)],
                                yc_v.at[pl.ds(off, ncw)])

        cw = [par_v[pl.ds(a * L, L)] for a in range(NTAP)]
        cbv = par_v[pl.ds(9 * L, L)]
        lbv = par_v[pl.ds(10 * L, L)]
        lane = lax.iota(jnp.int32, L)
        mask8 = lane < 8
        pre = []
        for j in range(CG):
            cc2 = j * L + lane
            bbj = cc2 // 24
            pre.append(bbj * (KP * C) + (cc2 - 24 * bbj) * 4)

        def chunk_body(ci, carry):
            mm0 = base0 + ci * KP
            for a in range(NTAP):
                for b2 in range(B):
                    woff = (a * B + b2) * ncw
                    for g in range(KP // L):
                        iv = a * m + mm0 + g * L + lane
                        mi = iv // NTAP
                        k = iv - mi * NTAP
                        xf = plsc.load_gather(xc_v, [mi - mi_al[a] + woff])
                        yf = plsc.load_gather(yc_v, [mi - mi_al[a] + woff])
                        xi = (xf * float(H)).astype(jnp.int32)
                        yi = (yf * float(W)).astype(jnp.int32)
                        xo = jnp.clip(xi + (k // 3 - 1), 0, H - 1)
                        yo = jnp.clip(yi + (k - k // 3 * 3 - 1), 0, W - 1)
                        idx_v[pl.ds((b2 * NTAP + a) * KP + g * L, L)] = (
                            xo * W + yo + b2 * HW)
            copies = []
            for b2 in range(B):
                boff = b2 * NTAP * KP
                for lo, ln in ((0, 128), (128, NTAP * KP - 128)):
                    copies.append(pltpu.async_copy(
                        table_h.at[idx_v.at[pl.ds(boff + lo, ln)]],
                        rows_v.at[pl.ds(boff + lo, ln)], sem))
            for cp in copies:
                cp.wait()

            def slot_body(t, inner):
                acc = [None] * B
                def addin(t_, v):
                    acc[t_] = v if acc[t_] is None else acc[t_] + v
                for b2 in range(B):
                    row = (b2 * NTAP + 4) * KP + t
                    for j in range(CG):
                        cen = rows_v[row, pl.ds(j * L, L)]
                        p = cen * lwp_v[pl.ds(b2 * C + j * L, L)]
                        band = (j * L) // 24
                        if j * L % 24 == 0 and (j * L + L) <= 24 * (band + 1):
                            addin(band, p)
                        elif (j * L + L) <= 24 * (band + 1):
                            addin(band, p)
                        else:
                            plo = jnp.where(mask8, p, 0.0)
                            addin(band, plo)
                            addin(band + 1, p - plo)
                acc = [a_ + lbv * (1.0 / L) for a_ in acc]
                conf = [jnp.sum(a_) for a_ in acc]
                cf = [jnp.full((L,), 1.0 - cs, jnp.float32) for cs in conf]
                coeff = [cf[0], jnp.where(mask8, cf[0], cf[1]), cf[1],
                         cf[2], jnp.where(mask8, cf[2], cf[3]), cf[3]]
                tof = t * C
                for b2 in range(B):
                    for j in range(CG):
                        pool = cbv
                        cen = None
                        for a in range(NTAP):
                            r = rows_v[(b2 * NTAP + a) * KP + t, pl.ds(j * L, L)]
                            if a == 4:
                                cen = r
                            pool = pool + cw[a] * r
                        val = (coeff[j] * pool + cen
                               + sp_v[pl.ds(b2 * C + j * L, L)])
                        plsc.store_scatter(out_v, [pre[j] + (tof + b2)], val)
                return inner

            lax.fori_loop(0, KP, slot_body, 0)
            for b2 in range(B):
                pltpu.sync_copy(
                    out_v.at[pl.ds(b2 * KP * C, KP * C)],
                    out_h.at[pl.ds((b2 * m + mm0) * C, KP * C)])
            return carry

        lax.fori_loop(0, n_chunks, chunk_body, 0)

    return body(table, xs, ys, par, lwp, sperm)


def kernel(original_kpts, segment, conv_w, conv_b, lin_w, lin_b, ca_w1, ca_w2):
    b, c, h, w = segment.shape
    m = original_kpts.shape[1]
    table, s3 = _tc_prep(segment, ca_w1, ca_w2)
    s = s3.reshape(b, c)
    pad = jnp.zeros((64,), jnp.float32)
    xs = jnp.concatenate([original_kpts[..., 0].reshape(-1), pad])
    ys = jnp.concatenate([original_kpts[..., 1].reshape(-1), pad])
    par = jnp.concatenate([
        jnp.repeat(conv_w.reshape(9), L),
        jnp.full((L,), conv_b[0], jnp.float32),
        jnp.full((L,), lin_b[0], jnp.float32),
    ])
    cb4 = c // b
    lwp = jnp.tile(lin_w.reshape(cb4, b).T, (1, b)).reshape(-1)
    sperm = jnp.transpose(s.reshape(b, cb4, b), (2, 0, 1)).reshape(-1)
    flat = _sc_combine(table, xs, ys, par, lwp, sperm, m)
    return flat.reshape(b, m, c)


# speedup vs baseline: 1.1034x; 1.0205x over previous
import functools

import jax
import jax.numpy as jnp
from jax import lax
from jax.experimental import pallas as pl
from jax.experimental.pallas import tpu as pltpu
from jax.experimental.pallas import tpu_sc as plsc

B = 4
C = 96
H = 384
W = 384
HW = H * W
T = 24576
L = 16
CG = C // L
KP = 16
NTAP = 9


def _prep_body(seg_ref, w1_ref, w2_ref, fmap_ref, s_ref, sum_ref, max_ref):
    t = pl.program_id(1)
    nt = pl.num_programs(1)
    blk = seg_ref[0].reshape(C, T)
    fmap_ref[...] = jnp.concatenate(
        [blk.T, jnp.zeros((T, 128 - C), jnp.float32)], axis=1)
    psum = jnp.sum(blk, axis=1)[None, :]
    pmax = jnp.max(blk, axis=1)[None, :]

    @pl.when(t == 0)
    def _init():
        sum_ref[...] = psum
        max_ref[...] = pmax

    @pl.when(t != 0)
    def _acc():
        sum_ref[...] += psum
        max_ref[...] = jnp.maximum(max_ref[...], pmax)

    @pl.when(t == nt - 1)
    def _mlp():
        avg = sum_ref[...] * (1.0 / HW)
        mxv = max_ref[...]
        w1 = w1_ref[...]
        w2 = w2_ref[...]

        def fc(v):
            hid = lax.dot_general(v, w1, (((1,), (1,)), ((), ())),
                                  preferred_element_type=jnp.float32)
            hid = jnp.maximum(hid, 0.0)
            return lax.dot_general(hid, w2, (((1,), (1,)), ((), ())),
                                   preferred_element_type=jnp.float32)

        s_ref[0] = jax.nn.sigmoid(fc(avg) + fc(mxv))


def _tc_prep(segment, ca_w1, ca_w2):
    b = segment.shape[0]
    nt = HW // T
    rows = T // W
    return pl.pallas_call(
        _prep_body,
        grid=(b, nt),
        in_specs=[
            pl.BlockSpec((1, C, rows, W), lambda i, t: (i, 0, t, 0)),
            pl.BlockSpec((C, C), lambda i, t: (0, 0)),
            pl.BlockSpec((C, C), lambda i, t: (0, 0)),
        ],
        out_specs=[
            pl.BlockSpec((T, 128), lambda i, t: (i * nt + t, 0)),
            pl.BlockSpec((1, 1, C), lambda i, t: (i, 0, 0)),
        ],
        out_shape=[
            jax.ShapeDtypeStruct((b * HW, 128), jnp.float32),
            jax.ShapeDtypeStruct((b, 1, C), jnp.float32),
        ],
        scratch_shapes=[
            pltpu.VMEM((1, C), jnp.float32),
            pltpu.VMEM((1, C), jnp.float32),
        ],
    )(segment, ca_w1, ca_w2)


def _sc_combine(table, xs, ys, par, lwp, sperm, m):
    info = plsc.get_sparse_core_info()
    nc, ns = info.num_cores, info.num_subcores
    nw = nc * ns
    slots_w = m // nw
    n_chunks = slots_w // KP
    ncw = 32
    mesh = plsc.VectorSubcoreMesh(core_axis_name="c", subcore_axis_name="s")

    @functools.partial(
        pl.kernel,
        mesh=mesh,
        out_type=jax.ShapeDtypeStruct((B * m * 128,), jnp.float32),
        scratch_types=[
            pltpu.VMEM((NTAP * B * ncw,), jnp.float32),
            pltpu.VMEM((NTAP * B * ncw,), jnp.float32),
            pltpu.VMEM((NTAP * B * KP,), jnp.int32),
            pltpu.VMEM((NTAP * B * KP, 128), jnp.float32),
            pltpu.VMEM((B * KP * 128,), jnp.float32),
            pltpu.VMEM((11 * L,), jnp.float32),
            pltpu.VMEM((B * C,), jnp.float32),
            pltpu.VMEM((B * C,), jnp.float32),
            pltpu.SemaphoreType.DMA,
        ],
        compiler_params=pltpu.CompilerParams(use_tc_tiling_on_sc=False,
                                             needs_layout_passes=False),
    )
    def body(table_h, xs_h, ys_h, par_h, lwp_h, sperm_h, out_h,
             xc_v, yc_v, idx_v, rows_v, out_v, par_v, lwp_v, sp_v, sem):
        wid = lax.axis_index("s") * nc + lax.axis_index("c")
        base0 = wid * slots_w
        pltpu.sync_copy(par_h, par_v)
        pltpu.sync_copy(lwp_h, lwp_v)
        pltpu.sync_copy(sperm_h, sp_v)

        mi_al = []
        for a in range(NTAP):
            lo = ((a * m + base0) // NTAP) // 8 * 8
            mi_al.append(lo)
            for b2 in range(B):
                off = (a * B + b2) * ncw
                pltpu.sync_copy(xs_h.at[pl.ds(b2 * m + lo, ncw)],
                                xc_v.at[pl.ds(off, ncw)])
                pltpu.sync_copy(ys_h.at[pl.ds(b2 * m + lo, ncw)],
                                yc_v.at[pl.ds(off, ncw)])

        cw = [par_v[pl.ds(a * L, L)] for a in range(NTAP)]
        cbv = par_v[pl.ds(9 * L, L)]
        lbv = par_v[pl.ds(10 * L, L)]
        lane = lax.iota(jnp.int32, L)
        mask8 = lane < 8
        pre = []
        for j in range(CG):
            cc2 = j * L + lane
            bbj = cc2 // 24
            pre.append(bbj * (KP * 128) + (cc2 - 24 * bbj) * 4)

        def chunk_body(ci, carry):
            mm0 = base0 + ci * KP
            for a in range(NTAP):
                for b2 in range(B):
                    woff = (a * B + b2) * ncw
                    for g in range(KP // L):
                        iv = a * m + mm0 + g * L + lane
                        mi = iv // NTAP
                        k = iv - mi * NTAP
                        xf = plsc.load_gather(xc_v, [mi - mi_al[a] + woff])
                        yf = plsc.load_gather(yc_v, [mi - mi_al[a] + woff])
                        xi = (xf * float(H)).astype(jnp.int32)
                        yi = (yf * float(W)).astype(jnp.int32)
                        xo = jnp.clip(xi + (k // 3 - 1), 0, H - 1)
                        yo = jnp.clip(yi + (k - k // 3 * 3 - 1), 0, W - 1)
                        idx_v[pl.ds((b2 * NTAP + a) * KP + g * L, L)] = (
                            xo * W + yo + b2 * HW)
            copies = []
            for b2 in range(B):
                boff = b2 * NTAP * KP
                for lo, ln in ((0, 128), (128, NTAP * KP - 128)):
                    copies.append(pltpu.async_copy(
                        table_h.at[idx_v.at[pl.ds(boff + lo, ln)]],
                        rows_v.at[pl.ds(boff + lo, ln)], sem))
            for cp in copies:
                cp.wait()

            def slot_body(t, inner):
                acc = [None] * B
                def addin(t_, v):
                    acc[t_] = v if acc[t_] is None else acc[t_] + v
                for b2 in range(B):
                    row = (b2 * NTAP + 4) * KP + t
                    for j in range(CG):
                        cen = rows_v[row, pl.ds(j * L, L)]
                        p = cen * lwp_v[pl.ds(b2 * C + j * L, L)]
                        band = (j * L) // 24
                        if j * L % 24 == 0 and (j * L + L) <= 24 * (band + 1):
                            addin(band, p)
                        elif (j * L + L) <= 24 * (band + 1):
                            addin(band, p)
                        else:
                            plo = jnp.where(mask8, p, 0.0)
                            addin(band, plo)
                            addin(band + 1, p - plo)
                acc = [a_ + lbv * (1.0 / L) for a_ in acc]
                conf = [jnp.sum(a_) for a_ in acc]
                cf = [jnp.full((L,), 1.0 - cs, jnp.float32) for cs in conf]
                coeff = [cf[0], jnp.where(mask8, cf[0], cf[1]), cf[1],
                         cf[2], jnp.where(mask8, cf[2], cf[3]), cf[3]]
                tof = t * 128
                for b2 in range(B):
                    for j in range(CG):
                        pool = cbv
                        cen = None
                        for a in range(NTAP):
                            r = rows_v[(b2 * NTAP + a) * KP + t, pl.ds(j * L, L)]
                            if a == 4:
                                cen = r
                            pool = pool + cw[a] * r
                        val = (coeff[j] * pool + cen
                               + sp_v[pl.ds(b2 * C + j * L, L)])
                        plsc.store_scatter(out_v, [pre[j] + (tof + b2)], val)
                return inner

            lax.fori_loop(0, KP, slot_body, 0)
            for b2 in range(B):
                pltpu.sync_copy(
                    out_v.at[pl.ds(b2 * KP * 128, KP * 128)],
                    out_h.at[pl.ds((b2 * m + mm0) * 128, KP * 128)])
            return carry

        lax.fori_loop(0, n_chunks, chunk_body, 0)

    return body(table, xs, ys, par, lwp, sperm)


def kernel(original_kpts, segment, conv_w, conv_b, lin_w, lin_b, ca_w1, ca_w2):
    b, c, h, w = segment.shape
    m = original_kpts.shape[1]
    table, s3 = _tc_prep(segment, ca_w1, ca_w2)
    s = s3.reshape(b, c)
    pad = jnp.zeros((64,), jnp.float32)
    xs = jnp.concatenate([original_kpts[..., 0].reshape(-1), pad])
    ys = jnp.concatenate([original_kpts[..., 1].reshape(-1), pad])
    par = jnp.concatenate([
        jnp.repeat(conv_w.reshape(9), L),
        jnp.full((L,), conv_b[0], jnp.float32),
        jnp.full((L,), lin_b[0], jnp.float32),
    ])
    cb4 = c // b
    lwp = jnp.tile(lin_w.reshape(cb4, b).T, (1, b)).reshape(-1)
    sperm = jnp.transpose(s.reshape(b, cb4, b), (2, 0, 1)).reshape(-1)
    flat = _sc_combine(table, xs, ys, par, lwp, sperm, m)
    return flat.reshape(b, m, 128)[..., :c]
